# baseline (speedup 1.0000x reference)
import functools

import jax
import jax.numpy as jnp
from jax import lax
from jax.experimental import pallas as pl
from jax.experimental.pallas import tpu as pltpu
from jax.experimental.pallas import tpu_sc as plsc

NC = 2
NS = 16
NW = NC * NS
CH = 128
K = 16
SCH = K * CH
TW = 16
BM = 3328


def _make_edge_pass(E: int, N: int, F: int):
    S = -(-E // (NW * SCH))
    Nacc = N + CH
    rpt = Nacc // NS
    mesh = plsc.VectorSubcoreMesh(
        core_axis_name="c", subcore_axis_name="s",
        num_cores=NC, num_subcores=NS)

    @functools.partial(
        pl.kernel,
        out_type=jax.ShapeDtypeStruct((NC, Nacc, TW), jnp.float32),
        mesh=mesh,
        compiler_params=pltpu.CompilerParams(
            needs_layout_passes=False, use_tc_tiling_on_sc=False),
        scratch_types=[
            pltpu.VMEM((N,), jnp.float32),
            pltpu.VMEM((K, CH), jnp.int32),
            pltpu.VMEM((K, CH), jnp.int32),
            pltpu.VMEM((SCH, TW), jnp.float32),
            pltpu.VMEM_SHARED((Nacc, TW), jnp.float32),
            pltpu.SemaphoreType.DMA,
        ],
    )
    def edge_pass(tbl_hbm, ad_hbm, src_hbm, dst_hbm, out_hbm,
                  ad_v, sidx_v, didx_v, rows_v, acc_sh, sem):
        c = lax.axis_index("c")
        s = lax.axis_index("s")
        w = s * NC + c

        def _zrow(i, _):
            rows_v[i] = jnp.zeros((TW,), jnp.float32)
            return 0
        lax.fori_loop(0, SCH, _zrow, 0)
        base = s * rpt
        off = 0
        while off < rpt:
            n = min(SCH, rpt - off)
            pltpu.sync_copy(rows_v.at[pl.ds(0, n)],
                            acc_sh.at[pl.ds(base + off, n)])
            off += n
        pltpu.sync_copy(ad_hbm, ad_v)
        plsc.subcore_barrier()

        iota16 = lax.iota(jnp.int32, 16)

        def _super(si, _):
            row0 = (w * S + si) * K
            pltpu.sync_copy(src_hbm.at[pl.ds(row0, K)], sidx_v)
            pltpu.sync_copy(dst_hbm.at[pl.ds(row0, K)], didx_v)
            for j in range(K):
                pltpu.async_copy(tbl_hbm.at[sidx_v.at[j]],
                                 rows_v.at[pl.ds(j * CH, CH)], sem)
            pltpu.make_async_copy(tbl_hbm.at[pl.ds(0, SCH)], rows_v, sem).wait()

            def _chunk(j, _):
                for g in range(CH // 16):
                    rids = j * CH + g * 16 + iota16
                    dvec = didx_v[j, pl.ds(g * 16, 16)]
                    a_s = plsc.load_gather(
                        rows_v, [rids, jnp.full((16,), TW - 1, jnp.int32)])
                    a_d = plsc.load_gather(ad_v, [jnp.minimum(dvec, N - 1)])
                    lg = a_s + a_d
                    ex = jnp.exp(jnp.where(lg >= 0, lg, 0.2 * lg))
                    plsc.store_scatter(
                        rows_v, [rids, jnp.zeros((16,), jnp.int32)], ex)
                    for f in range(1, F + 1):
                        cf = jnp.full((16,), f, jnp.int32)
                        v = plsc.load_gather(rows_v, [rids, cf])
                        plsc.store_scatter(rows_v, [rids, cf], v * ex)
                return 0
            lax.fori_loop(0, K, _chunk, 0)

            descs = [pltpu.async_copy(rows_v.at[pl.ds(j * CH, CH)],
                                      acc_sh.at[didx_v.at[j]], sem, add=True)
                     for j in range(K)]
            for d in descs:
                d.wait()
            return 0
        lax.fori_loop(0, S, _super, 0)

        plsc.subcore_barrier()
        off = 0
        while off < rpt:
            n = min(SCH, rpt - off)
            pltpu.sync_copy(acc_sh.at[pl.ds(base + off, n)],
                            out_hbm.at[c, pl.ds(base + off, n)])
            off += n

    return edge_pass, S, Nacc


def _leaky(x):
    return jnp.where(x >= 0, x, 0.2 * x)


def _prep1(x2d, W1, a1s, a1d):
    N = x2d.shape[0]
    bm = BM

    def body(x_ref, w_ref, as_ref, ad_ref, t_ref, adv_ref, sl_ref):
        x = x_ref[...]
        cs = jnp.sum(w_ref[...] * as_ref[...])
        cd = jnp.sum(w_ref[...] * ad_ref[...])
        al_s = x * cs
        al_d = x * cd
        t_ref[...] = jnp.concatenate(
            [jnp.ones((bm, 1), jnp.float32), x,
             jnp.zeros((bm, TW - 3), jnp.float32), al_s], axis=1)
        adv_ref[...] = al_d
        exs = jnp.exp(_leaky(al_s + al_d))
        sl_ref[...] = jnp.concatenate([exs, exs * x], axis=1)

    return pl.pallas_call(
        body,
        grid=(N // bm,),
        in_specs=[
            pl.BlockSpec((bm, 1), lambda i: (i, 0)),
            pl.BlockSpec((1, 8), lambda i: (0, 0)),
            pl.BlockSpec((1, 8), lambda i: (0, 0)),
            pl.BlockSpec((1, 8), lambda i: (0, 0)),
        ],
        out_specs=(
            pl.BlockSpec((bm, TW), lambda i: (i, 0)),
            pl.BlockSpec((bm, 1), lambda i: (i, 0)),
            pl.BlockSpec((bm, 2), lambda i: (i, 0)),
        ),
        out_shape=(
            jax.ShapeDtypeStruct((N, TW), jnp.float32),
            jax.ShapeDtypeStruct((N, 1), jnp.float32),
            jax.ShapeDtypeStruct((N, 2), jnp.float32),
        ),
    )(x2d, W1, a1s.reshape(1, 8), a1d.reshape(1, 8))


def _fin1_prep2(p0, p1, sl1, W1, b1, W2, a2s, a2d):
    N = p0.shape[0]
    bm = BM

    def body(p0_ref, p1_ref, sl_ref, w1_ref, b1_ref, w2_ref, as_ref, ad_ref,
             x1_ref, t_ref, adv_ref, sl2_ref):
        den = p0_ref[:, 0:1] + p1_ref[:, 0:1] + sl_ref[:, 0:1]
        num = p0_ref[:, 1:2] + p1_ref[:, 1:2] + sl_ref[:, 1:2]
        agg = num / (den + 1e-16)
        x1 = jnp.maximum(agg * w1_ref[...] + b1_ref[...], 0.0)
        x1_ref[...] = x1
        vs = jnp.sum(w2_ref[...] * as_ref[...], axis=1, keepdims=True)
        vd = jnp.sum(w2_ref[...] * ad_ref[...], axis=1, keepdims=True)
        al_s = jnp.dot(x1, vs, preferred_element_type=jnp.float32)
        al_d = jnp.dot(x1, vd, preferred_element_type=jnp.float32)
        t_ref[...] = jnp.concatenate(
            [jnp.ones((bm, 1), jnp.float32), x1,
             jnp.zeros((bm, TW - 10), jnp.float32), al_s], axis=1)
        adv_ref[...] = al_d
        exs = jnp.exp(_leaky(al_s + al_d))
        sl2_ref[...] = jnp.concatenate([exs, exs * x1], axis=1)

    return pl.pallas_call(
        body,
        grid=(N // bm,),
        in_specs=[
            pl.BlockSpec((bm, 2), lambda i: (i, 0)),
            pl.BlockSpec((bm, 2), lambda i: (i, 0)),
            pl.BlockSpec((bm, 2), lambda i: (i, 0)),
            pl.BlockSpec((1, 8), lambda i: (0, 0)),
            pl.BlockSpec((1, 8), lambda i: (0, 0)),
            pl.BlockSpec((8, 64), lambda i: (0, 0)),
            pl.BlockSpec((1, 64), lambda i: (0, 0)),
            pl.BlockSpec((1, 64), lambda i: (0, 0)),
        ],
        out_specs=(
            pl.BlockSpec((bm, 8), lambda i: (i, 0)),
            pl.BlockSpec((bm, TW), lambda i: (i, 0)),
            pl.BlockSpec((bm, 1), lambda i: (i, 0)),
            pl.BlockSpec((bm, 9), lambda i: (i, 0)),
        ),
        out_shape=(
            jax.ShapeDtypeStruct((N, 8), jnp.float32),
            jax.ShapeDtypeStruct((N, TW), jnp.float32),
            jax.ShapeDtypeStruct((N, 1), jnp.float32),
            jax.ShapeDtypeStruct((N, 9), jnp.float32),
        ),
    )(p0, p1, sl1, W1, b1.reshape(1, 8), W2,
      a2s.reshape(1, 64), a2d.reshape(1, 64))


def _fin2_prep3(p0, p1, sl2, W2, b2, W3, a3s, a3d):
    N = p0.shape[0]
    bm = BM

    def body(p0_ref, p1_ref, sl_ref, w2_ref, b2_ref, w3_ref, as_ref, ad_ref,
             x2_ref, t_ref, adv_ref, sl3_ref):
        den = p0_ref[:, 0:1] + p1_ref[:, 0:1] + sl_ref[:, 0:1]
        num = p0_ref[:, 1:9] + p1_ref[:, 1:9] + sl_ref[:, 1:9]
        agg = num / (den + 1e-16)
        x2 = jnp.maximum(
            jnp.dot(agg, w2_ref[...], preferred_element_type=jnp.float32)
            + b2_ref[...], 0.0)
        x2_ref[...] = x2
        h3 = jnp.dot(x2, w3_ref[...], preferred_element_type=jnp.float32)
        al_s = jnp.sum(h3 * as_ref[...], axis=1, keepdims=True)
        al_d = jnp.sum(h3 * ad_ref[...], axis=1, keepdims=True)
        t_ref[...] = jnp.concatenate(
            [jnp.ones((bm, 1), jnp.float32), h3,
             jnp.zeros((bm, TW - 11), jnp.float32), al_s], axis=1)
        adv_ref[...] = al_d
        exs = jnp.exp(_leaky(al_s + al_d))
        sl3_ref[...] = jnp.concatenate([exs, exs * h3], axis=1)

    return pl.pallas_call(
        body,
        grid=(N // bm,),
        in_specs=[
            pl.BlockSpec((bm, 9), lambda i: (i, 0)),
            pl.BlockSpec((bm, 9), lambda i: (i, 0)),
            pl.BlockSpec((bm, 9), lambda i: (i, 0)),
            pl.BlockSpec((8, 64), lambda i: (0, 0)),
            pl.BlockSpec((1, 64), lambda i: (0, 0)),
            pl.BlockSpec((64, 9), lambda i: (0, 0)),
            pl.BlockSpec((1, 9), lambda i: (0, 0)),
            pl.BlockSpec((1, 9), lambda i: (0, 0)),
        ],
        out_specs=(
            pl.BlockSpec((bm, 64), lambda i: (i, 0)),
            pl.BlockSpec((bm, TW), lambda i: (i, 0)),
            pl.BlockSpec((bm, 1), lambda i: (i, 0)),
            pl.BlockSpec((bm, 10), lambda i: (i, 0)),
        ),
        out_shape=(
            jax.ShapeDtypeStruct((N, 64), jnp.float32),
            jax.ShapeDtypeStruct((N, TW), jnp.float32),
            jax.ShapeDtypeStruct((N, 1), jnp.float32),
            jax.ShapeDtypeStruct((N, 10), jnp.float32),
        ),
    )(p0, p1, sl2, W2, b2.reshape(1, 64), W3,
      a3s.reshape(1, 9), a3d.reshape(1, 9))


def _fin3(p0, p1, sl3, b3):
    N = p0.shape[0]
    bm = BM

    def body(p0_ref, p1_ref, sl_ref, b3_ref, x3_ref):
        den = p0_ref[:, 0:1] + p1_ref[:, 0:1] + sl_ref[:, 0:1]
        num = p0_ref[:, 1:10] + p1_ref[:, 1:10] + sl_ref[:, 1:10]
        x3_ref[...] = jnp.maximum(num / (den + 1e-16) + b3_ref[...], 0.0)

    return pl.pallas_call(
        body,
        grid=(N // bm,),
        in_specs=[
            pl.BlockSpec((bm, 10), lambda i: (i, 0)),
            pl.BlockSpec((bm, 10), lambda i: (i, 0)),
            pl.BlockSpec((bm, 10), lambda i: (i, 0)),
            pl.BlockSpec((1, 9), lambda i: (0, 0)),
        ],
        out_specs=pl.BlockSpec((bm, 9), lambda i: (i, 0)),
        out_shape=jax.ShapeDtypeStruct((N, 9), jnp.float32),
    )(p0, p1, sl3, b3.reshape(1, 9))


def _pool_max(xa3d):
    B, G, Fc = xa3d.shape

    def body(x_ref, o_ref):
        m = x_ref[:, 0, :]
        for j in range(1, G):
            m = jnp.maximum(m, x_ref[:, j, :])
        o_ref[...] = m

    return pl.pallas_call(
        body,
        out_shape=jax.ShapeDtypeStruct((B, Fc), jnp.float32),
    )(xa3d)


def _pad_cols(a, newc, br, dtype=jnp.float32):
    R, C = a.shape

    def body(a_ref, o_ref):
        o_ref[...] = jnp.concatenate(
            [a_ref[...], jnp.zeros((br, newc - C), jnp.float32)],
            axis=1).astype(dtype)

    return pl.pallas_call(
        body,
        grid=(R // br,),
        in_specs=[pl.BlockSpec((br, C), lambda i: (i, 0))],
        out_specs=pl.BlockSpec((br, newc), lambda i: (i, 0)),
        out_shape=jax.ShapeDtypeStruct((R, newc), dtype),
    )(a)


def _pad_rows(a, newr, bc):
    R, C = a.shape

    def body(a_ref, o_ref):
        o_ref[...] = jnp.concatenate(
            [a_ref[...], jnp.zeros((newr - R, bc), jnp.float32)], axis=0)

    return pl.pallas_call(
        body,
        grid=(C // bc,),
        in_specs=[pl.BlockSpec((R, bc), lambda i: (0, i))],
        out_specs=pl.BlockSpec((newr, bc), lambda i: (0, i)),
        out_shape=jax.ShapeDtypeStruct((newr, C), jnp.float32),
    )(a)


def _mm(a, b, bias, relu, bm, bn):
    M, Ka = a.shape
    Kb, Nn = b.shape
    assert Ka == Kb and Nn % bn == 0 and M % bm == 0

    def body(a_ref, b_ref, bias_ref, o_ref):
        acc = jnp.dot(a_ref[...], b_ref[...],
                      preferred_element_type=jnp.float32)
        acc = acc + bias_ref[...]
        if relu:
            acc = jnp.maximum(acc, 0.0)
        o_ref[...] = acc

    return pl.pallas_call(
        body,
        grid=(M // bm, Nn // bn),
        in_specs=[
            pl.BlockSpec((bm, Ka), lambda i, j: (i, 0)),
            pl.BlockSpec((Kb, bn), lambda i, j: (0, j)),
            pl.BlockSpec((1, bn), lambda i, j: (0, j)),
        ],
        out_specs=pl.BlockSpec((bm, bn), lambda i, j: (i, j)),
        out_shape=jax.ShapeDtypeStruct((M, Nn), jnp.float32),
    )(a, b, bias.reshape(1, Nn))


def kernel(x, edge_index, batch, W1, a1s, a1d, b1, W2, a2s, a2d, b2,
           W3, a3s, a3d, b3, Wl1, bl1, Wl2, bl2, Wl3, bl3):
    N = x.shape[0]
    E = edge_index.shape[1]
    B = 1024
    npg = N // B

    src = edge_index[0].astype(jnp.int32)
    dst = edge_index[1].astype(jnp.int32)
    ep1, S1, Nacc = _make_edge_pass(E, N, 1)
    ep2, _, _ = _make_edge_pass(E, N, 8)
    ep3, _, _ = _make_edge_pass(E, N, 9)
    Ep = NW * S1 * SCH
    if Ep > E:
        pad = Ep - E
        src = jnp.concatenate([src, jnp.zeros((pad,), jnp.int32)])
        dst = jnp.concatenate([dst, jnp.full((pad,), N, jnp.int32)])
    src2 = src.reshape(-1, CH)
    dst2 = dst.reshape(-1, CH)

    x2d = x.reshape(N, 1)

    T1, ad1, sl1 = _prep1(x2d, W1, a1s, a1d)
    acc1 = ep1(T1, ad1.reshape(N), src2, dst2)
    x1, T2, ad2, sl2 = _fin1_prep2(
        acc1[0, :N, :2], acc1[1, :N, :2], sl1, W1, b1, W2, a2s, a2d)

    acc2 = ep2(T2, ad2.reshape(N), src2, dst2)
    x2, T3, ad3, sl3 = _fin2_prep3(
        acc2[0, :N, :9], acc2[1, :N, :9], sl2, W2, b2, W3, a3s, a3d)

    acc3 = ep3(T3, ad3.reshape(N), src2, dst2)
    x3 = _fin3(acc3[0, :N, :10], acc3[1, :N, :10], sl3, b3)

    xa = jnp.concatenate([x2d, x1, x2, x3], axis=1)
    pooled = _pool_max(xa.reshape(B, npg, 82))

    f = jnp.concatenate([
        x2d.reshape(B, npg), x1.reshape(B, npg * 8),
        x2.reshape(B, npg * 64), x3.reshape(B, npg * 9),
        pooled,
    ], axis=1)

    H1 = Wl1.shape[1]
    H1p = -(-H1 // 128) * 128
    Wl1p = _pad_cols(Wl1, H1p, 656, jnp.bfloat16)
    Wl2p = _pad_rows(Wl2, H1p, 256)
    Wl3p = _pad_cols(Wl3, 128, 256)
    bl1p = jnp.pad(bl1, (0, H1p - H1))
    bl3p = jnp.pad(bl3, (0, 128 - Wl3.shape[1]))

    g1 = _mm(f.astype(jnp.bfloat16), Wl1p, bl1p, True, 256, 640)
    g2 = _mm(g1, Wl2p, bl2, True, 256, 512)
    g3 = _mm(g2, Wl3p, bl3p, False, 256, 128)
    return g3[:, :Wl3.shape[1]]


# speedup vs baseline: 1.0565x; 1.0565x over previous
import functools

import jax
import jax.numpy as jnp
from jax import lax
from jax.experimental import pallas as pl
from jax.experimental.pallas import tpu as pltpu
from jax.experimental.pallas import tpu_sc as plsc

NC = 2
NS = 16
NW = NC * NS
CH = 128
K = 16
SCH = K * CH
TW = 16
BM = 3328


def _make_edge_pass(E: int, N: int, F: int):
    S = -(-E // (NW * SCH))
    Nacc = N + CH
    rpt = Nacc // NS
    mesh = plsc.VectorSubcoreMesh(
        core_axis_name="c", subcore_axis_name="s",
        num_cores=NC, num_subcores=NS)

    @functools.partial(
        pl.kernel,
        out_type=jax.ShapeDtypeStruct((NC, Nacc, TW), jnp.float32),
        mesh=mesh,
        compiler_params=pltpu.CompilerParams(
            needs_layout_passes=False, use_tc_tiling_on_sc=False),
        scratch_types=[
            pltpu.VMEM((N,), jnp.float32),
            pltpu.VMEM((K, CH), jnp.int32),
            pltpu.VMEM((K, CH), jnp.int32),
            pltpu.VMEM((SCH, TW), jnp.float32),
            pltpu.VMEM_SHARED((Nacc, TW), jnp.float32),
            pltpu.SemaphoreType.DMA,
        ],
    )
    def edge_pass(tbl_hbm, ad_hbm, src_hbm, dst_hbm, out_hbm,
                  ad_v, sidx_v, didx_v, rows_v, acc_sh, sem):
        c = lax.axis_index("c")
        s = lax.axis_index("s")
        w = s * NC + c

        def _zrow(i, _):
            rows_v[i] = jnp.zeros((TW,), jnp.float32)
            return 0
        lax.fori_loop(0, SCH, _zrow, 0)
        base = s * rpt
        off = 0
        while off < rpt:
            n = min(SCH, rpt - off)
            pltpu.sync_copy(rows_v.at[pl.ds(0, n)],
                            acc_sh.at[pl.ds(base + off, n)])
            off += n
        pltpu.sync_copy(ad_hbm, ad_v)
        plsc.subcore_barrier()

        iota16 = lax.iota(jnp.int32, 16)

        def _super(si, _):
            row0 = (w * S + si) * K
            pltpu.sync_copy(src_hbm.at[pl.ds(row0, K)], sidx_v)
            pltpu.sync_copy(dst_hbm.at[pl.ds(row0, K)], didx_v)
            for j in range(K):
                pltpu.async_copy(tbl_hbm.at[sidx_v.at[j]],
                                 rows_v.at[pl.ds(j * CH, CH)], sem)
            pltpu.make_async_copy(tbl_hbm.at[pl.ds(0, SCH)], rows_v, sem).wait()

            def _chunk(j, _):
                for g in range(CH // 16):
                    rids = j * CH + g * 16 + iota16
                    dvec = didx_v[j, pl.ds(g * 16, 16)]
                    a_s = plsc.load_gather(
                        rows_v, [rids, jnp.full((16,), TW - 1, jnp.int32)])
                    a_d = plsc.load_gather(ad_v, [jnp.minimum(dvec, N - 1)])
                    lg = a_s + a_d
                    ex = jnp.exp(jnp.where(lg >= 0, lg, 0.2 * lg))
                    plsc.store_scatter(
                        rows_v, [rids, jnp.zeros((16,), jnp.int32)], ex)
                    for f in range(1, F + 1):
                        cf = jnp.full((16,), f, jnp.int32)
                        v = plsc.load_gather(rows_v, [rids, cf])
                        plsc.store_scatter(rows_v, [rids, cf], v * ex)
                return 0
            lax.fori_loop(0, K, _chunk, 0)

            descs = [pltpu.async_copy(rows_v.at[pl.ds(j * CH, CH)],
                                      acc_sh.at[didx_v.at[j]], sem, add=True)
                     for j in range(K)]
            for d in descs:
                d.wait()
            return 0
        lax.fori_loop(0, S, _super, 0)

        plsc.subcore_barrier()
        off = 0
        while off < rpt:
            n = min(SCH, rpt - off)
            pltpu.sync_copy(acc_sh.at[pl.ds(base + off, n)],
                            out_hbm.at[c, pl.ds(base + off, n)])
            off += n

    return edge_pass, S, Nacc


def _leaky(x):
    return jnp.where(x >= 0, x, 0.2 * x)


def _prep1(x2d, W1, a1s, a1d):
    N = x2d.shape[0]
    bm = BM

    def body(x_ref, w_ref, as_ref, ad_ref, t_ref, adv_ref, sl_ref):
        x = x_ref[...]
        cs = jnp.sum(w_ref[...] * as_ref[...])
        cd = jnp.sum(w_ref[...] * ad_ref[...])
        al_s = x * cs
        al_d = x * cd
        t_ref[...] = jnp.concatenate(
            [jnp.ones((bm, 1), jnp.float32), x,
             jnp.zeros((bm, TW - 3), jnp.float32), al_s], axis=1)
        adv_ref[...] = al_d
        exs = jnp.exp(_leaky(al_s + al_d))
        sl_ref[...] = jnp.concatenate([exs, exs * x], axis=1)

    return pl.pallas_call(
        body,
        grid=(N // bm,),
        in_specs=[
            pl.BlockSpec((bm, 1), lambda i: (i, 0)),
            pl.BlockSpec((1, 8), lambda i: (0, 0)),
            pl.BlockSpec((1, 8), lambda i: (0, 0)),
            pl.BlockSpec((1, 8), lambda i: (0, 0)),
        ],
        out_specs=(
            pl.BlockSpec((bm, TW), lambda i: (i, 0)),
            pl.BlockSpec((bm, 1), lambda i: (i, 0)),
            pl.BlockSpec((bm, 2), lambda i: (i, 0)),
        ),
        out_shape=(
            jax.ShapeDtypeStruct((N, TW), jnp.float32),
            jax.ShapeDtypeStruct((N, 1), jnp.float32),
            jax.ShapeDtypeStruct((N, 2), jnp.float32),
        ),
    )(x2d, W1, a1s.reshape(1, 8), a1d.reshape(1, 8))


def _fin1_prep2(acc, N, sl1, W1, b1, W2, a2s, a2d):
    bm = BM

    def body(p0_ref, p1_ref, sl_ref, w1_ref, b1_ref, w2_ref, as_ref, ad_ref,
             x1_ref, t_ref, adv_ref, sl2_ref):
        p0 = p0_ref[0]
        p1 = p1_ref[0]
        den = p0[:, 0:1] + p1[:, 0:1] + sl_ref[:, 0:1]
        num = p0[:, 1:2] + p1[:, 1:2] + sl_ref[:, 1:2]
        agg = num / (den + 1e-16)
        x1 = jnp.maximum(agg * w1_ref[...] + b1_ref[...], 0.0)
        x1_ref[...] = x1
        vs = jnp.sum(w2_ref[...] * as_ref[...], axis=1, keepdims=True)
        vd = jnp.sum(w2_ref[...] * ad_ref[...], axis=1, keepdims=True)
        al_s = jnp.dot(x1, vs, preferred_element_type=jnp.float32)
        al_d = jnp.dot(x1, vd, preferred_element_type=jnp.float32)
        t_ref[...] = jnp.concatenate(
            [jnp.ones((bm, 1), jnp.float32), x1,
             jnp.zeros((bm, TW - 10), jnp.float32), al_s], axis=1)
        adv_ref[...] = al_d
        exs = jnp.exp(_leaky(al_s + al_d))
        sl2_ref[...] = jnp.concatenate([exs, exs * x1], axis=1)

    return pl.pallas_call(
        body,
        grid=(N // bm,),
        in_specs=[
            pl.BlockSpec((1, bm, TW), lambda i: (0, i, 0)),
            pl.BlockSpec((1, bm, TW), lambda i: (1, i, 0)),
            pl.BlockSpec((bm, 2), lambda i: (i, 0)),
            pl.BlockSpec((1, 8), lambda i: (0, 0)),
            pl.BlockSpec((1, 8), lambda i: (0, 0)),
            pl.BlockSpec((8, 64), lambda i: (0, 0)),
            pl.BlockSpec((1, 64), lambda i: (0, 0)),
            pl.BlockSpec((1, 64), lambda i: (0, 0)),
        ],
        out_specs=(
            pl.BlockSpec((bm, 8), lambda i: (i, 0)),
            pl.BlockSpec((bm, TW), lambda i: (i, 0)),
            pl.BlockSpec((bm, 1), lambda i: (i, 0)),
            pl.BlockSpec((bm, 9), lambda i: (i, 0)),
        ),
        out_shape=(
            jax.ShapeDtypeStruct((N, 8), jnp.float32),
            jax.ShapeDtypeStruct((N, TW), jnp.float32),
            jax.ShapeDtypeStruct((N, 1), jnp.float32),
            jax.ShapeDtypeStruct((N, 9), jnp.float32),
        ),
    )(acc, acc, sl1, W1, b1.reshape(1, 8), W2,
      a2s.reshape(1, 64), a2d.reshape(1, 64))


def _fin2_prep3(acc, N, sl2, W2, b2, W3, a3s, a3d):
    bm = BM

    def body(p0_ref, p1_ref, sl_ref, w2_ref, b2_ref, w3_ref, as_ref, ad_ref,
             x2_ref, t_ref, adv_ref, sl3_ref):
        p0 = p0_ref[0]
        p1 = p1_ref[0]
        den = p0[:, 0:1] + p1[:, 0:1] + sl_ref[:, 0:1]
        num = p0[:, 1:9] + p1[:, 1:9] + sl_ref[:, 1:9]
        agg = num / (den + 1e-16)
        x2 = jnp.maximum(
            jnp.dot(agg, w2_ref[...], preferred_element_type=jnp.float32)
            + b2_ref[...], 0.0)
        x2_ref[...] = x2
        h3 = jnp.dot(x2, w3_ref[...], preferred_element_type=jnp.float32)
        al_s = jnp.sum(h3 * as_ref[...], axis=1, keepdims=True)
        al_d = jnp.sum(h3 * ad_ref[...], axis=1, keepdims=True)
        t_ref[...] = jnp.concatenate(
            [jnp.ones((bm, 1), jnp.float32), h3,
             jnp.zeros((bm, TW - 11), jnp.float32), al_s], axis=1)
        adv_ref[...] = al_d
        exs = jnp.exp(_leaky(al_s + al_d))
        sl3_ref[...] = jnp.concatenate([exs, exs * h3], axis=1)

    return pl.pallas_call(
        body,
        grid=(N // bm,),
        in_specs=[
            pl.BlockSpec((1, bm, TW), lambda i: (0, i, 0)),
            pl.BlockSpec((1, bm, TW), lambda i: (1, i, 0)),
            pl.BlockSpec((bm, 9), lambda i: (i, 0)),
            pl.BlockSpec((8, 64), lambda i: (0, 0)),
            pl.BlockSpec((1, 64), lambda i: (0, 0)),
            pl.BlockSpec((64, 9), lambda i: (0, 0)),
            pl.BlockSpec((1, 9), lambda i: (0, 0)),
            pl.BlockSpec((1, 9), lambda i: (0, 0)),
        ],
        out_specs=(
            pl.BlockSpec((bm, 64), lambda i: (i, 0)),
            pl.BlockSpec((bm, TW), lambda i: (i, 0)),
            pl.BlockSpec((bm, 1), lambda i: (i, 0)),
            pl.BlockSpec((bm, 10), lambda i: (i, 0)),
        ),
        out_shape=(
            jax.ShapeDtypeStruct((N, 64), jnp.float32),
            jax.ShapeDtypeStruct((N, TW), jnp.float32),
            jax.ShapeDtypeStruct((N, 1), jnp.float32),
            jax.ShapeDtypeStruct((N, 10), jnp.float32),
        ),
    )(acc, acc, sl2, W2, b2.reshape(1, 64), W3,
      a3s.reshape(1, 9), a3d.reshape(1, 9))


def _fin3(acc, N, sl3, b3):
    bm = BM

    def body(p0_ref, p1_ref, sl_ref, b3_ref, x3_ref):
        p0 = p0_ref[0]
        p1 = p1_ref[0]
        den = p0[:, 0:1] + p1[:, 0:1] + sl_ref[:, 0:1]
        num = p0[:, 1:10] + p1[:, 1:10] + sl_ref[:, 1:10]
        x3_ref[...] = jnp.maximum(num / (den + 1e-16) + b3_ref[...], 0.0)

    return pl.pallas_call(
        body,
        grid=(N // bm,),
        in_specs=[
            pl.BlockSpec((1, bm, TW), lambda i: (0, i, 0)),
            pl.BlockSpec((1, bm, TW), lambda i: (1, i, 0)),
            pl.BlockSpec((bm, 10), lambda i: (i, 0)),
            pl.BlockSpec((1, 9), lambda i: (0, 0)),
        ],
        out_specs=pl.BlockSpec((bm, 9), lambda i: (i, 0)),
        out_shape=jax.ShapeDtypeStruct((N, 9), jnp.float32),
    )(acc, acc, sl3, b3.reshape(1, 9))


def _pool_max(xa3d):
    B, G, Fc = xa3d.shape

    def body(x_ref, o_ref):
        m = x_ref[:, 0, :]
        for j in range(1, G):
            m = jnp.maximum(m, x_ref[:, j, :])
        o_ref[...] = m

    return pl.pallas_call(
        body,
        out_shape=jax.ShapeDtypeStruct((B, Fc), jnp.float32),
    )(xa3d)


def _pad_cols(a, newc, br, dtype=jnp.float32):
    R, C = a.shape

    def body(a_ref, o_ref):
        o_ref[...] = jnp.concatenate(
            [a_ref[...], jnp.zeros((br, newc - C), jnp.float32)],
            axis=1).astype(dtype)

    return pl.pallas_call(
        body,
        grid=(R // br,),
        in_specs=[pl.BlockSpec((br, C), lambda i: (i, 0))],
        out_specs=pl.BlockSpec((br, newc), lambda i: (i, 0)),
        out_shape=jax.ShapeDtypeStruct((R, newc), dtype),
    )(a)


def _pad_rows(a, newr, bc):
    R, C = a.shape

    def body(a_ref, o_ref):
        o_ref[...] = jnp.concatenate(
            [a_ref[...], jnp.zeros((newr - R, bc), jnp.float32)], axis=0)

    return pl.pallas_call(
        body,
        grid=(C // bc,),
        in_specs=[pl.BlockSpec((R, bc), lambda i: (0, i))],
        out_specs=pl.BlockSpec((newr, bc), lambda i: (0, i)),
        out_shape=jax.ShapeDtypeStruct((newr, C), jnp.float32),
    )(a)


def _mm(a, b, bias, relu, bm, bn):
    M, Ka = a.shape
    Kb, Nn = b.shape
    assert Ka == Kb and Nn % bn == 0 and M % bm == 0

    def body(a_ref, b_ref, bias_ref, o_ref):
        acc = jnp.dot(a_ref[...], b_ref[...],
                      preferred_element_type=jnp.float32)
        acc = acc + bias_ref[...]
        if relu:
            acc = jnp.maximum(acc, 0.0)
        o_ref[...] = acc

    return pl.pallas_call(
        body,
        grid=(M // bm, Nn // bn),
        in_specs=[
            pl.BlockSpec((bm, Ka), lambda i, j: (i, 0)),
            pl.BlockSpec((Kb, bn), lambda i, j: (0, j)),
            pl.BlockSpec((1, bn), lambda i, j: (0, j)),
        ],
        out_specs=pl.BlockSpec((bm, bn), lambda i, j: (i, j)),
        out_shape=jax.ShapeDtypeStruct((M, Nn), jnp.float32),
    )(a, b, bias.reshape(1, Nn))


def kernel(x, edge_index, batch, W1, a1s, a1d, b1, W2, a2s, a2d, b2,
           W3, a3s, a3d, b3, Wl1, bl1, Wl2, bl2, Wl3, bl3):
    N = x.shape[0]
    E = edge_index.shape[1]
    B = 1024
    npg = N // B

    src = edge_index[0].astype(jnp.int32)
    dst = edge_index[1].astype(jnp.int32)
    ep1, S1, Nacc = _make_edge_pass(E, N, 1)
    ep2, _, _ = _make_edge_pass(E, N, 8)
    ep3, _, _ = _make_edge_pass(E, N, 9)
    Ep = NW * S1 * SCH
    if Ep > E:
        pad = Ep - E
        src = jnp.concatenate([src, jnp.zeros((pad,), jnp.int32)])
        dst = jnp.concatenate([dst, jnp.full((pad,), N, jnp.int32)])
    src2 = src.reshape(-1, CH)
    dst2 = dst.reshape(-1, CH)

    x2d = x.reshape(N, 1)

    T1, ad1, sl1 = _prep1(x2d, W1, a1s, a1d)
    acc1 = ep1(T1, ad1.reshape(N), src2, dst2)
    x1, T2, ad2, sl2 = _fin1_prep2(acc1, N, sl1, W1, b1, W2, a2s, a2d)

    acc2 = ep2(T2, ad2.reshape(N), src2, dst2)
    x2, T3, ad3, sl3 = _fin2_prep3(acc2, N, sl2, W2, b2, W3, a3s, a3d)

    acc3 = ep3(T3, ad3.reshape(N), src2, dst2)
    x3 = _fin3(acc3, N, sl3, b3)

    xa = jnp.concatenate([x2d, x1, x2, x3], axis=1)
    pooled = _pool_max(xa.reshape(B, npg, 82))

    f = jnp.concatenate([
        x2d.reshape(B, npg), x1.reshape(B, npg * 8),
        x2.reshape(B, npg * 64), x3.reshape(B, npg * 9),
        pooled,
    ], axis=1)

    H1 = Wl1.shape[1]
    H1p = -(-H1 // 128) * 128
    Wl1p = _pad_cols(Wl1, H1p, 656, jnp.bfloat16)
    Wl2p = _pad_rows(Wl2, H1p, 256)
    Wl3p = _pad_cols(Wl3, 128, 256)
    bl1p = jnp.pad(bl1, (0, H1p - H1))
    bl3p = jnp.pad(bl3, (0, 128 - Wl3.shape[1]))

    g1 = _mm(f.astype(jnp.bfloat16), Wl1p, bl1p, True, 256, 640)
    g2 = _mm(g1, Wl2p, bl2, True, 256, 512)
    g3 = _mm(g2, Wl3p, bl3p, False, 256, 128)
    return g3[:, :Wl3.shape[1]]


# speedup vs baseline: 1.1188x; 1.0590x over previous
import functools

import jax
import jax.numpy as jnp
from jax import lax
from jax.experimental import pallas as pl
from jax.experimental.pallas import tpu as pltpu
from jax.experimental.pallas import tpu_sc as plsc

NC = 2
NS = 16
NW = NC * NS
CH = 128
K = 16
SCH = K * CH
TW = 16
BM = 3328
LB_NUM, LB_DEN = 6, 10


def _make_edge_pass(E: int, N: int, F: int):
    S = -(-E // (NW * SCH))
    S0 = (2 * S * LB_NUM) // LB_DEN
    S1 = 2 * S - S0
    Nacc = N + CH
    rpt = Nacc // NS
    mesh = plsc.VectorSubcoreMesh(
        core_axis_name="c", subcore_axis_name="s",
        num_cores=NC, num_subcores=NS)

    @functools.partial(
        pl.kernel,
        out_type=jax.ShapeDtypeStruct((NC, Nacc, TW), jnp.float32),
        mesh=mesh,
        compiler_params=pltpu.CompilerParams(
            needs_layout_passes=False, use_tc_tiling_on_sc=False),
        scratch_types=[
            pltpu.VMEM((N,), jnp.float32),
            pltpu.VMEM((K, CH), jnp.int32),
            pltpu.VMEM((K, CH), jnp.int32),
            pltpu.VMEM((SCH, TW), jnp.float32),
            pltpu.VMEM_SHARED((Nacc, TW), jnp.float32),
            pltpu.SemaphoreType.DMA,
        ],
    )
    def edge_pass(tbl_hbm, ad_hbm, src_hbm, dst_hbm, out_hbm,
                  ad_v, sidx_v, didx_v, rows_v, acc_sh, sem):
        c = lax.axis_index("c")
        s = lax.axis_index("s")
        def _zrow(i, _):
            rows_v[i] = jnp.zeros((TW,), jnp.float32)
            return 0
        lax.fori_loop(0, SCH, _zrow, 0)
        base = s * rpt
        off = 0
        while off < rpt:
            n = min(SCH, rpt - off)
            pltpu.sync_copy(rows_v.at[pl.ds(0, n)],
                            acc_sh.at[pl.ds(base + off, n)])
            off += n
        pltpu.sync_copy(ad_hbm, ad_v)
        plsc.subcore_barrier()

        iota16 = lax.iota(jnp.int32, 16)
        nsup = jnp.where(c == 0, S0, S1)
        sbase = jnp.where(c == 0, s * S0, 16 * S0 + s * S1)

        def _super(si, _):
            row0 = (sbase + si) * K
            pltpu.sync_copy(src_hbm.at[pl.ds(row0, K)], sidx_v)
            pltpu.sync_copy(dst_hbm.at[pl.ds(row0, K)], didx_v)
            for j in range(K):
                pltpu.async_copy(tbl_hbm.at[sidx_v.at[j]],
                                 rows_v.at[pl.ds(j * CH, CH)], sem)
            pltpu.make_async_copy(tbl_hbm.at[pl.ds(0, SCH)], rows_v, sem).wait()

            def _chunk(j, _):
                for g in range(CH // 16):
                    rids = j * CH + g * 16 + iota16
                    dvec = didx_v[j, pl.ds(g * 16, 16)]
                    a_s = plsc.load_gather(
                        rows_v, [rids, jnp.full((16,), TW - 1, jnp.int32)])
                    a_d = plsc.load_gather(ad_v, [jnp.minimum(dvec, N - 1)])
                    lg = a_s + a_d
                    ex = jnp.exp(jnp.where(lg >= 0, lg, 0.2 * lg))
                    plsc.store_scatter(
                        rows_v, [rids, jnp.zeros((16,), jnp.int32)], ex)
                    for f in range(1, F + 1):
                        cf = jnp.full((16,), f, jnp.int32)
                        v = plsc.load_gather(rows_v, [rids, cf])
                        plsc.store_scatter(rows_v, [rids, cf], v * ex)
                return 0
            lax.fori_loop(0, K, _chunk, 0)

            descs = [pltpu.async_copy(rows_v.at[pl.ds(j * CH, CH)],
                                      acc_sh.at[didx_v.at[j]], sem, add=True)
                     for j in range(K)]
            for d in descs:
                d.wait()
            return 0
        lax.fori_loop(0, nsup, _super, 0)

        plsc.subcore_barrier()
        off = 0
        while off < rpt:
            n = min(SCH, rpt - off)
            pltpu.sync_copy(acc_sh.at[pl.ds(base + off, n)],
                            out_hbm.at[c, pl.ds(base + off, n)])
            off += n

    return edge_pass, S, Nacc


def _leaky(x):
    return jnp.where(x >= 0, x, 0.2 * x)


def _prep1(x2d, W1, a1s, a1d):
    N = x2d.shape[0]
    bm = BM

    def body(x_ref, w_ref, as_ref, ad_ref, t_ref, adv_ref, sl_ref):
        x = x_ref[...]
        cs = jnp.sum(w_ref[...] * as_ref[...])
        cd = jnp.sum(w_ref[...] * ad_ref[...])
        al_s = x * cs
        al_d = x * cd
        t_ref[...] = jnp.concatenate(
            [jnp.ones((bm, 1), jnp.float32), x,
             jnp.zeros((bm, TW - 3), jnp.float32), al_s], axis=1)
        adv_ref[...] = al_d
        exs = jnp.exp(_leaky(al_s + al_d))
        sl_ref[...] = jnp.concatenate([exs, exs * x], axis=1)

    return pl.pallas_call(
        body,
        grid=(N // bm,),
        in_specs=[
            pl.BlockSpec((bm, 1), lambda i: (i, 0)),
            pl.BlockSpec((1, 8), lambda i: (0, 0)),
            pl.BlockSpec((1, 8), lambda i: (0, 0)),
            pl.BlockSpec((1, 8), lambda i: (0, 0)),
        ],
        out_specs=(
            pl.BlockSpec((bm, TW), lambda i: (i, 0)),
            pl.BlockSpec((bm, 1), lambda i: (i, 0)),
            pl.BlockSpec((bm, 2), lambda i: (i, 0)),
        ),
        out_shape=(
            jax.ShapeDtypeStruct((N, TW), jnp.float32),
            jax.ShapeDtypeStruct((N, 1), jnp.float32),
            jax.ShapeDtypeStruct((N, 2), jnp.float32),
        ),
    )(x2d, W1, a1s.reshape(1, 8), a1d.reshape(1, 8))


def _fin1_prep2(acc, N, sl1, W1, b1, W2, a2s, a2d):
    bm = BM

    def body(p0_ref, p1_ref, sl_ref, w1_ref, b1_ref, w2_ref, as_ref, ad_ref,
             x1_ref, t_ref, adv_ref, sl2_ref):
        p0 = p0_ref[0]
        p1 = p1_ref[0]
        den = p0[:, 0:1] + p1[:, 0:1] + sl_ref[:, 0:1]
        num = p0[:, 1:2] + p1[:, 1:2] + sl_ref[:, 1:2]
        agg = num / (den + 1e-16)
        x1 = jnp.maximum(agg * w1_ref[...] + b1_ref[...], 0.0)
        x1_ref[...] = x1
        vs = jnp.sum(w2_ref[...] * as_ref[...], axis=1, keepdims=True)
        vd = jnp.sum(w2_ref[...] * ad_ref[...], axis=1, keepdims=True)
        al_s = jnp.dot(x1, vs, preferred_element_type=jnp.float32)
        al_d = jnp.dot(x1, vd, preferred_element_type=jnp.float32)
        t_ref[...] = jnp.concatenate(
            [jnp.ones((bm, 1), jnp.float32), x1,
             jnp.zeros((bm, TW - 10), jnp.float32), al_s], axis=1)
        adv_ref[...] = al_d
        exs = jnp.exp(_leaky(al_s + al_d))
        sl2_ref[...] = jnp.concatenate([exs, exs * x1], axis=1)

    return pl.pallas_call(
        body,
        grid=(N // bm,),
        in_specs=[
            pl.BlockSpec((1, bm, TW), lambda i: (0, i, 0)),
            pl.BlockSpec((1, bm, TW), lambda i: (1, i, 0)),
            pl.BlockSpec((bm, 2), lambda i: (i, 0)),
            pl.BlockSpec((1, 8), lambda i: (0, 0)),
            pl.BlockSpec((1, 8), lambda i: (0, 0)),
            pl.BlockSpec((8, 64), lambda i: (0, 0)),
            pl.BlockSpec((1, 64), lambda i: (0, 0)),
            pl.BlockSpec((1, 64), lambda i: (0, 0)),
        ],
        out_specs=(
            pl.BlockSpec((bm, 8), lambda i: (i, 0)),
            pl.BlockSpec((bm, TW), lambda i: (i, 0)),
            pl.BlockSpec((bm, 1), lambda i: (i, 0)),
            pl.BlockSpec((bm, 9), lambda i: (i, 0)),
        ),
        out_shape=(
            jax.ShapeDtypeStruct((N, 8), jnp.float32),
            jax.ShapeDtypeStruct((N, TW), jnp.float32),
            jax.ShapeDtypeStruct((N, 1), jnp.float32),
            jax.ShapeDtypeStruct((N, 9), jnp.float32),
        ),
    )(acc, acc, sl1, W1, b1.reshape(1, 8), W2,
      a2s.reshape(1, 64), a2d.reshape(1, 64))


def _fin2_prep3(acc, N, sl2, W2, b2, W3, a3s, a3d):
    bm = BM

    def body(p0_ref, p1_ref, sl_ref, w2_ref, b2_ref, w3_ref, as_ref, ad_ref,
             x2_ref, t_ref, adv_ref, sl3_ref):
        p0 = p0_ref[0]
        p1 = p1_ref[0]
        den = p0[:, 0:1] + p1[:, 0:1] + sl_ref[:, 0:1]
        num = p0[:, 1:9] + p1[:, 1:9] + sl_ref[:, 1:9]
        agg = num / (den + 1e-16)
        x2 = jnp.maximum(
            jnp.dot(agg, w2_ref[...], preferred_element_type=jnp.float32)
            + b2_ref[...], 0.0)
        x2_ref[...] = x2
        h3 = jnp.dot(x2, w3_ref[...], preferred_element_type=jnp.float32)
        al_s = jnp.sum(h3 * as_ref[...], axis=1, keepdims=True)
        al_d = jnp.sum(h3 * ad_ref[...], axis=1, keepdims=True)
        t_ref[...] = jnp.concatenate(
            [jnp.ones((bm, 1), jnp.float32), h3,
             jnp.zeros((bm, TW - 11), jnp.float32), al_s], axis=1)
        adv_ref[...] = al_d
        exs = jnp.exp(_leaky(al_s + al_d))
        sl3_ref[...] = jnp.concatenate([exs, exs * h3], axis=1)

    return pl.pallas_call(
        body,
        grid=(N // bm,),
        in_specs=[
            pl.BlockSpec((1, bm, TW), lambda i: (0, i, 0)),
            pl.BlockSpec((1, bm, TW), lambda i: (1, i, 0)),
            pl.BlockSpec((bm, 9), lambda i: (i, 0)),
            pl.BlockSpec((8, 64), lambda i: (0, 0)),
            pl.BlockSpec((1, 64), lambda i: (0, 0)),
            pl.BlockSpec((64, 9), lambda i: (0, 0)),
            pl.BlockSpec((1, 9), lambda i: (0, 0)),
            pl.BlockSpec((1, 9), lambda i: (0, 0)),
        ],
        out_specs=(
            pl.BlockSpec((bm, 64), lambda i: (i, 0)),
            pl.BlockSpec((bm, TW), lambda i: (i, 0)),
            pl.BlockSpec((bm, 1), lambda i: (i, 0)),
            pl.BlockSpec((bm, 10), lambda i: (i, 0)),
        ),
        out_shape=(
            jax.ShapeDtypeStruct((N, 64), jnp.float32),
            jax.ShapeDtypeStruct((N, TW), jnp.float32),
            jax.ShapeDtypeStruct((N, 1), jnp.float32),
            jax.ShapeDtypeStruct((N, 10), jnp.float32),
        ),
    )(acc, acc, sl2, W2, b2.reshape(1, 64), W3,
      a3s.reshape(1, 9), a3d.reshape(1, 9))


def _fin3(acc, N, sl3, b3):
    bm = BM

    def body(p0_ref, p1_ref, sl_ref, b3_ref, x3_ref):
        p0 = p0_ref[0]
        p1 = p1_ref[0]
        den = p0[:, 0:1] + p1[:, 0:1] + sl_ref[:, 0:1]
        num = p0[:, 1:10] + p1[:, 1:10] + sl_ref[:, 1:10]
        x3_ref[...] = jnp.maximum(num / (den + 1e-16) + b3_ref[...], 0.0)

    return pl.pallas_call(
        body,
        grid=(N // bm,),
        in_specs=[
            pl.BlockSpec((1, bm, TW), lambda i: (0, i, 0)),
            pl.BlockSpec((1, bm, TW), lambda i: (1, i, 0)),
            pl.BlockSpec((bm, 10), lambda i: (i, 0)),
            pl.BlockSpec((1, 9), lambda i: (0, 0)),
        ],
        out_specs=pl.BlockSpec((bm, 9), lambda i: (i, 0)),
        out_shape=jax.ShapeDtypeStruct((N, 9), jnp.float32),
    )(acc, acc, sl3, b3.reshape(1, 9))


def _pool_max(xa3d):
    B, G, Fc = xa3d.shape

    def body(x_ref, o_ref):
        m = x_ref[:, 0, :]
        for j in range(1, G):
            m = jnp.maximum(m, x_ref[:, j, :])
        o_ref[...] = m

    return pl.pallas_call(
        body,
        out_shape=jax.ShapeDtypeStruct((B, Fc), jnp.float32),
    )(xa3d)


def _pad_cols(a, newc, br, dtype=jnp.float32):
    R, C = a.shape

    def body(a_ref, o_ref):
        o_ref[...] = jnp.concatenate(
            [a_ref[...], jnp.zeros((br, newc - C), jnp.float32)],
            axis=1).astype(dtype)

    return pl.pallas_call(
        body,
        grid=(R // br,),
        in_specs=[pl.BlockSpec((br, C), lambda i: (i, 0))],
        out_specs=pl.BlockSpec((br, newc), lambda i: (i, 0)),
        out_shape=jax.ShapeDtypeStruct((R, newc), dtype),
    )(a)


def _pad_rows(a, newr, bc):
    R, C = a.shape

    def body(a_ref, o_ref):
        o_ref[...] = jnp.concatenate(
            [a_ref[...], jnp.zeros((newr - R, bc), jnp.float32)], axis=0)

    return pl.pallas_call(
        body,
        grid=(C // bc,),
        in_specs=[pl.BlockSpec((R, bc), lambda i: (0, i))],
        out_specs=pl.BlockSpec((newr, bc), lambda i: (0, i)),
        out_shape=jax.ShapeDtypeStruct((newr, C), jnp.float32),
    )(a)


def _mm(a, b, bias, relu, bm, bn):
    M, Ka = a.shape
    Kb, Nn = b.shape
    assert Ka == Kb and Nn % bn == 0 and M % bm == 0

    def body(a_ref, b_ref, bias_ref, o_ref):
        acc = jnp.dot(a_ref[...], b_ref[...],
                      preferred_element_type=jnp.float32)
        acc = acc + bias_ref[...]
        if relu:
            acc = jnp.maximum(acc, 0.0)
        o_ref[...] = acc

    return pl.pallas_call(
        body,
        grid=(M // bm, Nn // bn),
        in_specs=[
            pl.BlockSpec((bm, Ka), lambda i, j: (i, 0)),
            pl.BlockSpec((Kb, bn), lambda i, j: (0, j)),
            pl.BlockSpec((1, bn), lambda i, j: (0, j)),
        ],
        out_specs=pl.BlockSpec((bm, bn), lambda i, j: (i, j)),
        out_shape=jax.ShapeDtypeStruct((M, Nn), jnp.float32),
    )(a, b, bias.reshape(1, Nn))


def kernel(x, edge_index, batch, W1, a1s, a1d, b1, W2, a2s, a2d, b2,
           W3, a3s, a3d, b3, Wl1, bl1, Wl2, bl2, Wl3, bl3):
    N = x.shape[0]
    E = edge_index.shape[1]
    B = 1024
    npg = N // B

    src = edge_index[0].astype(jnp.int32)
    dst = edge_index[1].astype(jnp.int32)
    ep1, S1, Nacc = _make_edge_pass(E, N, 1)
    ep2, _, _ = _make_edge_pass(E, N, 8)
    ep3, _, _ = _make_edge_pass(E, N, 9)
    Ep = NW * S1 * SCH
    if Ep > E:
        pad = Ep - E
        src = jnp.concatenate([src, jnp.zeros((pad,), jnp.int32)])
        dst = jnp.concatenate([dst, jnp.full((pad,), N, jnp.int32)])
    src2 = src.reshape(-1, CH)
    dst2 = dst.reshape(-1, CH)

    x2d = x.reshape(N, 1)

    T1, ad1, sl1 = _prep1(x2d, W1, a1s, a1d)
    acc1 = ep1(T1, ad1.reshape(N), src2, dst2)
    x1, T2, ad2, sl2 = _fin1_prep2(acc1, N, sl1, W1, b1, W2, a2s, a2d)

    acc2 = ep2(T2, ad2.reshape(N), src2, dst2)
    x2, T3, ad3, sl3 = _fin2_prep3(acc2, N, sl2, W2, b2, W3, a3s, a3d)

    acc3 = ep3(T3, ad3.reshape(N), src2, dst2)
    x3 = _fin3(acc3, N, sl3, b3)

    xa = jnp.concatenate([x2d, x1, x2, x3], axis=1)
    pooled = _pool_max(xa.reshape(B, npg, 82))

    f = jnp.concatenate([
        x2d.reshape(B, npg), x1.reshape(B, npg * 8),
        x2.reshape(B, npg * 64), x3.reshape(B, npg * 9),
        pooled,
    ], axis=1)

    H1 = Wl1.shape[1]
    H1p = -(-H1 // 128) * 128
    Wl1p = _pad_cols(Wl1, H1p, 656, jnp.bfloat16)
    Wl2p = _pad_rows(Wl2, H1p, 256)
    Wl3p = _pad_cols(Wl3, 128, 256)
    bl1p = jnp.pad(bl1, (0, H1p - H1))
    bl3p = jnp.pad(bl3, (0, 128 - Wl3.shape[1]))

    g1 = _mm(f.astype(jnp.bfloat16), Wl1p, bl1p, True, 256, 640)
    g2 = _mm(g1, Wl2p, bl2, True, 256, 512)
    g3 = _mm(g2, Wl3p, bl3p, False, 256, 128)
    return g3[:, :Wl3.shape[1]]
